# baseline (speedup 1.0000x reference)
import jax
import jax.numpy as jnp
from jax.experimental import pallas as pl
from jax.experimental.pallas import tpu as pltpu

B, H, D = 64, 32, 64
NUM_SLOTS = 128
NG = 3
NS = 3


def _body(slot_ref, perm_ref, ncopy_ref, cache_hbm, q_ref, k_ref, v_ref,
          slope_ref, outq_ref, newc_hbm, gbuf, sbuf, sem_copy, sem_in,
          sem_out):
    ncopy = ncopy_ref[0]
    ratio = jnp.exp(-slope_ref[0])

    def copy_body(i, carry):
        s = perm_ref[i]
        pltpu.make_async_copy(cache_hbm.at[s], newc_hbm.at[s],
                              sem_copy).start()
        return carry
    jax.lax.fori_loop(0, ncopy, copy_body, 0)

    for j in range(NG):
        pltpu.make_async_copy(cache_hbm.at[slot_ref[j]], gbuf.at[j],
                              sem_in.at[j]).start()

    def step(b, carry):
        jg = jax.lax.rem(b, NG)
        js = jax.lax.rem(b, NS)
        slot_b = slot_ref[b]
        pltpu.make_async_copy(cache_hbm.at[slot_b], gbuf.at[jg],
                              sem_in.at[jg]).wait()

        @pl.when(b >= NS)
        def _wait_store():
            pltpu.make_async_copy(sbuf.at[js], newc_hbm.at[slot_b],
                                  sem_out.at[js]).wait()

        kv_old = gbuf[jg]
        k3 = k_ref[b, :, 0, :]
        v3 = v_ref[b, :, 0, :]
        q3 = q_ref[b, :, 0, :]
        kv_new = (k3[:, :, None] * v3[:, None, :]
                  + ratio[:, None, None] * kv_old)
        sbuf[js] = kv_new
        outq_ref[b, :, 0, :] = jnp.sum(q3[:, :, None] * kv_new, axis=1)
        pltpu.make_async_copy(sbuf.at[js], newc_hbm.at[slot_b],
                              sem_out.at[js]).start()

        @pl.when(b + NG < B)
        def _next_gather():
            pltpu.make_async_copy(cache_hbm.at[slot_ref[b + NG]],
                                  gbuf.at[jg], sem_in.at[jg]).start()
        return carry
    jax.lax.fori_loop(0, B, step, 0)

    for j in range(NS):
        pltpu.make_async_copy(sbuf.at[j], newc_hbm.at[slot_ref[0]],
                              sem_out.at[j]).wait()

    def copy_wait(i, carry):
        s = perm_ref[i]
        pltpu.make_async_copy(cache_hbm.at[s], newc_hbm.at[s],
                              sem_copy).wait()
        return carry
    jax.lax.fori_loop(0, ncopy, copy_wait, 0)


def kernel(q, k, v, kv_caches, slope_rate, slot_idx):
    slot_idx = slot_idx.astype(jnp.int32)
    touched = jnp.zeros((NUM_SLOTS,), jnp.int32).at[slot_idx].set(1)
    perm = jnp.argsort(touched, stable=True).astype(jnp.int32)
    ncopy = (NUM_SLOTS - jnp.sum(touched)).reshape(1)
    slope2 = slope_rate.reshape(1, H)

    grid_spec = pltpu.PrefetchScalarGridSpec(
        num_scalar_prefetch=3,
        grid=(1,),
        in_specs=[
            pl.BlockSpec(memory_space=pltpu.MemorySpace.HBM),
            pl.BlockSpec((B, H, 1, D), lambda i, *_: (0, 0, 0, 0)),
            pl.BlockSpec((B, H, 1, D), lambda i, *_: (0, 0, 0, 0)),
            pl.BlockSpec((B, H, 1, D), lambda i, *_: (0, 0, 0, 0)),
            pl.BlockSpec((1, H), lambda i, *_: (0, 0)),
        ],
        out_specs=[
            pl.BlockSpec((B, H, 1, D), lambda i, *_: (0, 0, 0, 0)),
            pl.BlockSpec(memory_space=pltpu.MemorySpace.HBM),
        ],
        scratch_shapes=[
            pltpu.VMEM((NG, H, D, D), jnp.float32),
            pltpu.VMEM((NS, H, D, D), jnp.float32),
            pltpu.SemaphoreType.DMA,
            pltpu.SemaphoreType.DMA((NG,)),
            pltpu.SemaphoreType.DMA((NS,)),
        ],
    )
    output, new_cache = pl.pallas_call(
        _body,
        grid_spec=grid_spec,
        out_shape=[
            jax.ShapeDtypeStruct((B, H, 1, D), jnp.float32),
            jax.ShapeDtypeStruct((NUM_SLOTS, H, D, D), jnp.float32),
        ],
    )(slot_idx, perm, ncopy, kv_caches, q, k, v, slope2)
    return output, new_cache


# speedup vs baseline: 6.9408x; 6.9408x over previous
import jax
import jax.numpy as jnp
from jax.experimental import pallas as pl
from jax.experimental.pallas import tpu as pltpu

B, H, D = 64, 32, 64
NUM_SLOTS = 128
BS = 4


def _slot_kernel(inv_ref, cache_ref, q_ref, k_ref, v_ref, slope_ref,
                 newc_ref, out_ref):
    s = pl.program_id(0)
    ratio = jnp.exp(-slope_ref[0])
    kv_old = cache_ref[...]

    for j in range(BS):
        b = inv_ref[BS * s + j]
        bc = jnp.maximum(b, 0)
        k3 = k_ref[bc, :, 0, :]
        v3 = v_ref[bc, :, 0, :]
        q3 = q_ref[bc, :, 0, :]
        kvo = kv_old[j]
        kv_new = (k3[:, :, None] * v3[:, None, :]
                  + ratio[:, None, None] * kvo)
        newc_ref[j] = jnp.where(b >= 0, kv_new, kvo)
        out_ref[BS * s + j, :, 0, :] = jnp.sum(
            q3[:, :, None] * kv_new, axis=1)


def kernel(q, k, v, kv_caches, slope_rate, slot_idx):
    slot_idx = slot_idx.astype(jnp.int32)
    inv = jnp.full((NUM_SLOTS,), -1, jnp.int32).at[slot_idx].set(
        jnp.arange(B, dtype=jnp.int32))
    slope2 = slope_rate.reshape(1, H)

    grid_spec = pltpu.PrefetchScalarGridSpec(
        num_scalar_prefetch=1,
        grid=(NUM_SLOTS // BS,),
        in_specs=[
            pl.BlockSpec((BS, H, D, D), lambda s, inv: (s, 0, 0, 0)),
            pl.BlockSpec((B, H, 1, D), lambda s, inv: (0, 0, 0, 0)),
            pl.BlockSpec((B, H, 1, D), lambda s, inv: (0, 0, 0, 0)),
            pl.BlockSpec((B, H, 1, D), lambda s, inv: (0, 0, 0, 0)),
            pl.BlockSpec((1, H), lambda s, inv: (0, 0)),
        ],
        out_specs=[
            pl.BlockSpec((BS, H, D, D), lambda s, inv: (s, 0, 0, 0)),
            pl.BlockSpec((NUM_SLOTS, H, 1, D), lambda s, inv: (0, 0, 0, 0)),
        ],
    )
    new_cache, out_s = pl.pallas_call(
        _slot_kernel,
        grid_spec=grid_spec,
        out_shape=[
            jax.ShapeDtypeStruct((NUM_SLOTS, H, D, D), jnp.float32),
            jax.ShapeDtypeStruct((NUM_SLOTS, H, 1, D), jnp.float32),
        ],
    )(inv, kv_caches, q, k, v, slope2)
    output = jnp.take(out_s, slot_idx, axis=0)
    return output, new_cache


# speedup vs baseline: 7.0149x; 1.0107x over previous
import jax
import jax.numpy as jnp
from jax.experimental import pallas as pl
from jax.experimental.pallas import tpu as pltpu

B, H, D = 64, 32, 64
NUM_SLOTS = 128
BS = 4


def _slot_kernel(inv_ref, cache_ref, q_ref, k_ref, v_ref, slope_ref,
                 newc_ref, out_ref):
    s = pl.program_id(0)
    ratio = jnp.exp(-slope_ref[0])
    kv_old = cache_ref[...]

    for j in range(BS):
        b = inv_ref[BS * s + j]
        kvo = kv_old[j]

        @pl.when(b >= 0)
        def _update(b=b, kvo=kvo, j=j):
            k3 = k_ref[b, :, 0, :]
            v3 = v_ref[b, :, 0, :]
            q3 = q_ref[b, :, 0, :]
            kv_new = (k3[:, :, None] * v3[:, None, :]
                      + ratio[:, None, None] * kvo)
            newc_ref[j] = kv_new
            out_ref[b, :, 0, :] = jnp.sum(q3[:, :, None] * kv_new, axis=1)

        @pl.when(b < 0)
        def _copy(kvo=kvo, j=j):
            newc_ref[j] = kvo


def kernel(q, k, v, kv_caches, slope_rate, slot_idx):
    slot_idx = slot_idx.astype(jnp.int32)
    inv = jnp.full((NUM_SLOTS,), -1, jnp.int32).at[slot_idx].set(
        jnp.arange(B, dtype=jnp.int32))
    slope2 = slope_rate.reshape(1, H)

    grid_spec = pltpu.PrefetchScalarGridSpec(
        num_scalar_prefetch=1,
        grid=(NUM_SLOTS // BS,),
        in_specs=[
            pl.BlockSpec((BS, H, D, D), lambda s, inv: (s, 0, 0, 0)),
            pl.BlockSpec((B, H, 1, D), lambda s, inv: (0, 0, 0, 0)),
            pl.BlockSpec((B, H, 1, D), lambda s, inv: (0, 0, 0, 0)),
            pl.BlockSpec((B, H, 1, D), lambda s, inv: (0, 0, 0, 0)),
            pl.BlockSpec((1, H), lambda s, inv: (0, 0)),
        ],
        out_specs=[
            pl.BlockSpec((BS, H, D, D), lambda s, inv: (s, 0, 0, 0)),
            pl.BlockSpec((B, H, 1, D), lambda s, inv: (0, 0, 0, 0)),
        ],
    )
    new_cache, output = pl.pallas_call(
        _slot_kernel,
        grid_spec=grid_spec,
        out_shape=[
            jax.ShapeDtypeStruct((NUM_SLOTS, H, D, D), jnp.float32),
            jax.ShapeDtypeStruct((B, H, 1, D), jnp.float32),
        ],
    )(inv, kv_caches, q, k, v, slope2)
    return output, new_cache
